# baseline (speedup 1.0000x reference)
import functools

import jax
import jax.numpy as jnp
from jax import lax
from jax.experimental import pallas as pl
from jax.experimental.pallas import tpu as pltpu
from jax.experimental.pallas import tpu_sc as plsc

_B, _L, _V, _D = 16384, 200, 1000000, 32
_H = 150
_NC, _NS = 2, 16
_NW = _NC * _NS
_EPW = _B // _NW
_CH = 64
_L0, _L1 = 128, 72
_TCOLS = _V // 128
_TAILW = _V - 128 * _TCOLS
_XBLK = _B // 128

_MESH = dict(core_axis_name="c", subcore_axis_name="s")


def _iota16():
  return lax.iota(jnp.int32, 16)


def _repack_sc(tabT, xT, tail16):
  @functools.partial(
      pl.kernel,
      mesh=plsc.VectorSubcoreMesh(**_MESH),
      compiler_params=pltpu.CompilerParams(
          use_tc_tiling_on_sc=True, needs_layout_passes=False),
      out_type=(
          jax.ShapeDtypeStruct((_V // 4, 128), jnp.float32),
          jax.ShapeDtypeStruct((2 * _B, 128), jnp.int32),
      ),
      scratch_types=[
          pltpu.VMEM((2, 32, 128), jnp.float32),
          pltpu.VMEM((2, 32, 128), jnp.float32),
          pltpu.VMEM((2, 200, 128), jnp.int32),
          pltpu.VMEM((128, 128), jnp.int32),
          pltpu.SemaphoreType.DMA,
          pltpu.SemaphoreType.DMA,
          pltpu.SemaphoreType.DMA,
          pltpu.SemaphoreType.DMA,
          pltpu.SemaphoreType.DMA,
      ],
  )
  def body(tab_hbm, x_hbm, tail_hbm, packed_hbm, xfp_hbm, tin, tout, xin,
           xout, sem_ia, sem_ib, sem_oa, sem_ob, sem_x):
    wid = lax.axis_index("s") * _NC + lax.axis_index("c")

    def x_in(m, buf):
      pltpu.async_copy(
          x_hbm.at[:, pl.ds((wid + _NW * m) * 128, 128)], xin.at[buf],
          sem_x)

    def x_in_wait(buf):
      pltpu.make_async_copy(
          x_hbm.at[:, pl.ds(0, 128)], xin.at[buf], sem_x).wait()

    def x_block(m, buf):
      bb = wid + _NW * m
      for h in range(2):
        npiece = 8 if h == 0 else 5

        def xrow(b, carry):
          for k in range(npiece):
            ls = jnp.minimum(128 * h + 16 * k + _iota16(), _L - 1)
            vals = plsc.load_gather(
                xin.at[buf], [ls, jnp.full((16,), b, jnp.int32)])
            xout[b, pl.ds(16 * k, 16)] = vals
          return carry

        lax.fori_loop(0, 128, xrow, 0, unroll=8)
        pltpu.sync_copy(xout, xfp_hbm.at[pl.ds(h * _B + bb * 128, 128)])

    x_in(0, 0)
    for m in range(4):
      if m + 1 < 4:
        x_in(m + 1, (m + 1) % 2)
      x_in_wait(m % 2)
      x_block(m, m % 2)

    def t_in(tc, buf, sem):
      pltpu.async_copy(
          tab_hbm.at[:, pl.ds(128 * tc, 128)], tin.at[buf], sem)

    def t_in_wait(buf, sem):
      pltpu.make_async_copy(
          tab_hbm.at[:, pl.ds(0, 128)], tin.at[buf], sem).wait()

    def t_out(tc, buf, sem):
      pltpu.async_copy(
          tout.at[buf], packed_hbm.at[pl.ds(32 * tc, 32)], sem)

    def t_out_wait(buf, sem):
      pltpu.make_async_copy(
          tout.at[buf], packed_hbm.at[pl.ds(0, 32)], sem).wait()

    def t_block(buf):
      def trow(r, carry):
        for k in range(8):
          d = (16 if k % 2 else 0) + _iota16()
          col = jnp.full((16,), 4 * r + k // 2, jnp.int32)
          vals = plsc.load_gather(tin.at[buf], [d, col])
          tout[buf, r, pl.ds(16 * k, 16)] = vals
        return carry

      lax.fori_loop(0, 32, trow, 0, unroll=8)

    nfull = jnp.where(wid < 4, 245, 244)
    npairs = 122
    t_in(wid, 0, sem_ia)

    def t_pair(p, carry):
      k0 = 2 * p
      t_in(wid + _NW * (k0 + 1), 1, sem_ib)

      @pl.when(p >= 1)
      def _():
        t_out_wait(0, sem_oa)

      t_in_wait(0, sem_ia)
      t_block(0)
      t_out(wid + _NW * k0, 0, sem_oa)

      @pl.when(k0 + 2 < nfull)
      def _():
        t_in(wid + _NW * (k0 + 2), 0, sem_ia)

      @pl.when(p >= 1)
      def _():
        t_out_wait(1, sem_ob)

      t_in_wait(1, sem_ib)
      t_block(1)
      t_out(wid + _NW * (k0 + 1), 1, sem_ob)
      return carry

    lax.fori_loop(0, npairs, t_pair, 0)

    @pl.when(wid < 4)
    def _():
      t_out_wait(0, sem_oa)
      t_in_wait(0, sem_ia)
      t_block(0)
      t_out(wid + _NW * 244, 0, sem_oa)

    t_out_wait(0, sem_oa)
    t_out_wait(1, sem_ob)

    @pl.when(wid == 4)
    def _():
      pltpu.sync_copy(tail_hbm, packed_hbm.at[pl.ds(32 * _TCOLS,
                                                    _TAILW // 4)])

  return body(tabT, xT, tail16)


def _pool_sc(xfp, tab_lin):
  @functools.partial(
      pl.kernel,
      mesh=plsc.VectorSubcoreMesh(**_MESH),
      compiler_params=pltpu.CompilerParams(use_tc_tiling_on_sc=False),
      out_type=jax.ShapeDtypeStruct((_B, _D), jnp.float32),
      scratch_types=[
          pltpu.VMEM((2, _CH, 128), jnp.int32),
          pltpu.VMEM((2, _L0, _D), jnp.float32),
          pltpu.VMEM((2, _L1, _D), jnp.float32),
          pltpu.VMEM((_CH, _D), jnp.float32),
          pltpu.SemaphoreType.DMA,
          pltpu.SemaphoreType.DMA,
      ],
  )
  def body(x_hbm, tab_hbm, out_hbm, idx_v, rows_a, rows_b, pool_v,
           sem_a, sem_b):
    wid = lax.axis_index("s") * _NC + lax.axis_index("c")
    base = wid * _EPW

    def fire(j, buf, sem):
      pltpu.async_copy(tab_hbm.at[idx_v.at[0, j]], rows_a.at[buf], sem)
      pltpu.async_copy(tab_hbm.at[idx_v.at[1, j, pl.ds(0, _L1)]],
                       rows_b.at[buf], sem)

    def drain(buf, sem):
      pltpu.make_async_copy(
          tab_hbm.at[idx_v.at[0, 0]], rows_a.at[buf], sem).wait()
      pltpu.make_async_copy(
          tab_hbm.at[idx_v.at[1, 0, pl.ds(0, _L1)]], rows_b.at[buf],
          sem).wait()

    def accum(j, buf):
      def row2_body(r, accs):
        a0, a1 = accs
        a0 = a0 + rows_a[buf, r, pl.ds(0, 16)]
        a0 = a0 + rows_b[buf, r, pl.ds(0, 16)]
        a1 = a1 + rows_a[buf, r, pl.ds(16, 16)]
        a1 = a1 + rows_b[buf, r, pl.ds(16, 16)]
        return (a0, a1)

      def row1_body(r, accs):
        a0, a1 = accs
        a0 = a0 + rows_a[buf, r, pl.ds(0, 16)]
        a1 = a1 + rows_a[buf, r, pl.ds(16, 16)]
        return (a0, a1)

      z = jnp.zeros((16,), jnp.float32)
      accs = lax.fori_loop(0, _L1, row2_body, (z, z))
      a0, a1 = lax.fori_loop(_L1, _L0, row1_body, accs)
      pool_v[j, pl.ds(0, 16)] = a0 * (1.0 / _L)
      pool_v[j, pl.ds(16, 16)] = a1 * (1.0 / _L)

    def chunk_body(ci, carry):
      cbase = base + ci * _CH
      pltpu.sync_copy(x_hbm.at[0, pl.ds(cbase, _CH)], idx_v.at[0])
      pltpu.sync_copy(x_hbm.at[1, pl.ds(cbase, _CH)], idx_v.at[1])
      fire(0, 0, sem_a)

      def pair_body(p, carry2):
        j0 = 2 * p
        fire(j0 + 1, 1, sem_b)
        drain(0, sem_a)
        accum(j0, 0)

        @pl.when(j0 + 2 < _CH)
        def _():
          fire(j0 + 2, 0, sem_a)

        drain(1, sem_b)
        accum(j0 + 1, 1)
        return carry2

      lax.fori_loop(0, _CH // 2, pair_body, 0)
      pltpu.sync_copy(pool_v, out_hbm.at[pl.ds(cbase, _CH)])
      return carry

    lax.fori_loop(0, _EPW // _CH, chunk_body, 0)

  return body(xfp, tab_lin)


def _mlp_tc(pooled, W1, b1, W2, b2, W3, b3):
  bb = 2048

  def body(p_ref, w1_ref, b1_ref, w2_ref, b2_ref, w3_ref, b3_ref, o_ref):
    h = jnp.dot(p_ref[...], w1_ref[...], preferred_element_type=jnp.float32)
    h = jnp.maximum(h + b1_ref[...], 0.0)
    h = jnp.dot(h, w2_ref[...], preferred_element_type=jnp.float32)
    h = jnp.maximum(h + b2_ref[...], 0.0)
    o_ref[...] = (
        jnp.dot(h, w3_ref[...], preferred_element_type=jnp.float32)
        + b3_ref[...]
    )

  return pl.pallas_call(
      body,
      grid=(_B // bb,),
      in_specs=[
          pl.BlockSpec((bb, _D), lambda i: (i, 0)),
          pl.BlockSpec((_D, _H), lambda i: (0, 0)),
          pl.BlockSpec((1, _H), lambda i: (0, 0)),
          pl.BlockSpec((_H, _H), lambda i: (0, 0)),
          pl.BlockSpec((1, _H), lambda i: (0, 0)),
          pl.BlockSpec((_H, 1), lambda i: (0, 0)),
          pl.BlockSpec((1, 1), lambda i: (0, 0)),
      ],
      out_specs=pl.BlockSpec((bb, 1), lambda i: (i, 0)),
      out_shape=jax.ShapeDtypeStruct((_B, 1), jnp.float32),
  )(pooled, W1, b1.reshape(1, _H), W2, b2.reshape(1, _H), W3,
    b3.reshape(1, 1))


@jax.jit
def _run(x, table, W1, b1, W2, b2, W3, b3):
  tail16 = table[128 * _TCOLS:].reshape(_TAILW // 4, 128)
  packed, xfp = _repack_sc(table.T, x.astype(jnp.int32).T, tail16)
  tab_lin = packed.reshape(_V, _D)
  pooled = _pool_sc(xfp.reshape(2, _B, 128), tab_lin)
  return _mlp_tc(pooled, W1, b1, W2, b2, W3, b3)


def kernel(x, table, W1, b1, W2, b2, W3, b3):
  return _run(x, table, W1, b1, W2, b2, W3, b3)


# speedup vs baseline: 1.8885x; 1.8885x over previous
import functools

import jax
import jax.numpy as jnp
from jax import lax
from jax.experimental import pallas as pl
from jax.experimental.pallas import tpu as pltpu
from jax.experimental.pallas import tpu_sc as plsc

_B, _L, _V, _D = 16384, 200, 1000000, 32
_H = 150
_NC, _NS = 2, 16
_NW = _NC * _NS
_EPW = _B // _NW
_CH = 64
_BV = 8192
_BP = _BV // 4
_NBLK = (_V + _BV - 1) // _BV
_L0, _L1 = 128, 72

_MESH = dict(core_axis_name="c", subcore_axis_name="s")


def _phi(v):
  return (v & -_BV) | ((v & (_BP - 1)) << 2) | ((v >> 11) & 3)


def _repack_table_tc(tabT):
  def body(i_ref, o_ref):
    for c in range(4):
      o_ref[:, 32 * c:32 * (c + 1)] = jnp.transpose(
          i_ref[:, _BP * c:_BP * (c + 1)])

  return pl.pallas_call(
      body,
      grid=(_NBLK,),
      in_specs=[pl.BlockSpec((_D, _BV), lambda i: (0, i))],
      out_specs=pl.BlockSpec((_BP, 128), lambda i: (i, 0)),
      out_shape=jax.ShapeDtypeStruct((_NBLK * _BP, 128), jnp.float32),
  )(tabT)


def _repack_x_sc(xT):
  @functools.partial(
      pl.kernel,
      mesh=plsc.VectorSubcoreMesh(**_MESH),
      compiler_params=pltpu.CompilerParams(
          use_tc_tiling_on_sc=False, needs_layout_passes=False),
      out_type=jax.ShapeDtypeStruct((2 * _B, 128), jnp.int32),
      scratch_types=[
          pltpu.VMEM((2, _L, 128), jnp.int32),
          pltpu.VMEM((128, 128), jnp.int32),
          pltpu.SemaphoreType.DMA,
      ],
  )
  def body(x_hbm, xfp_hbm, xin, xout, sem_x):
    wid = lax.axis_index("s") * _NC + lax.axis_index("c")
    iota = lax.iota(jnp.int32, 16)

    def x_in(m, buf):
      pltpu.async_copy(
          x_hbm.at[:, pl.ds((wid + _NW * m) * 128, 128)], xin.at[buf],
          sem_x)

    def x_in_wait(buf):
      pltpu.make_async_copy(
          x_hbm.at[:, pl.ds(0, 128)], xin.at[buf], sem_x).wait()

    def x_block(m, buf):
      bb = wid + _NW * m
      for h in range(2):
        npiece = 8 if h == 0 else 5

        def xrow(b, carry):
          col = jnp.full((16,), b, jnp.int32)
          for k in range(npiece):
            ls = jnp.minimum(128 * h + 16 * k + iota, _L - 1)
            vals = plsc.load_gather(xin.at[buf], [ls, col])
            xout[b, pl.ds(16 * k, 16)] = _phi(vals)
          return carry

        lax.fori_loop(0, 128, xrow, 0, unroll=4)
        pltpu.sync_copy(xout, xfp_hbm.at[pl.ds(h * _B + bb * 128, 128)])

    x_in(0, 0)
    for m in range(4):
      if m + 1 < 4:
        x_in(m + 1, (m + 1) % 2)
      x_in_wait(m % 2)
      x_block(m, m % 2)

  return body(xT)


def _pool_sc(xfp, tab_lin):
  @functools.partial(
      pl.kernel,
      mesh=plsc.VectorSubcoreMesh(**_MESH),
      compiler_params=pltpu.CompilerParams(use_tc_tiling_on_sc=False),
      out_type=jax.ShapeDtypeStruct((_B, _D), jnp.float32),
      scratch_types=[
          pltpu.VMEM((2, _CH, 128), jnp.int32),
          pltpu.VMEM((2, _L0, _D), jnp.float32),
          pltpu.VMEM((2, _L1, _D), jnp.float32),
          pltpu.VMEM((_CH, _D), jnp.float32),
          pltpu.SemaphoreType.DMA,
          pltpu.SemaphoreType.DMA,
      ],
  )
  def body(x_hbm, tab_hbm, out_hbm, idx_v, rows_a, rows_b, pool_v,
           sem_a, sem_b):
    wid = lax.axis_index("s") * _NC + lax.axis_index("c")
    base = wid * _EPW

    def fire(j, buf, sem):
      pltpu.async_copy(tab_hbm.at[idx_v.at[0, j]], rows_a.at[buf], sem)
      pltpu.async_copy(tab_hbm.at[idx_v.at[1, j, pl.ds(0, _L1)]],
                       rows_b.at[buf], sem)

    def drain(buf, sem):
      pltpu.make_async_copy(
          tab_hbm.at[idx_v.at[0, 0]], rows_a.at[buf], sem).wait()
      pltpu.make_async_copy(
          tab_hbm.at[idx_v.at[1, 0, pl.ds(0, _L1)]], rows_b.at[buf],
          sem).wait()

    def accum(j, buf):
      def row2_body(r, accs):
        a0, a1 = accs
        a0 = a0 + rows_a[buf, r, pl.ds(0, 16)]
        a0 = a0 + rows_b[buf, r, pl.ds(0, 16)]
        a1 = a1 + rows_a[buf, r, pl.ds(16, 16)]
        a1 = a1 + rows_b[buf, r, pl.ds(16, 16)]
        return (a0, a1)

      def row1_body(r, accs):
        a0, a1 = accs
        a0 = a0 + rows_a[buf, r, pl.ds(0, 16)]
        a1 = a1 + rows_a[buf, r, pl.ds(16, 16)]
        return (a0, a1)

      z = jnp.zeros((16,), jnp.float32)
      accs = lax.fori_loop(0, _L1, row2_body, (z, z))
      a0, a1 = lax.fori_loop(_L1, _L0, row1_body, accs)
      pool_v[j, pl.ds(0, 16)] = a0 * (1.0 / _L)
      pool_v[j, pl.ds(16, 16)] = a1 * (1.0 / _L)

    def chunk_body(ci, carry):
      cbase = base + ci * _CH
      pltpu.sync_copy(x_hbm.at[0, pl.ds(cbase, _CH)], idx_v.at[0])
      pltpu.sync_copy(x_hbm.at[1, pl.ds(cbase, _CH)], idx_v.at[1])
      fire(0, 0, sem_a)

      def pair_body(p, carry2):
        j0 = 2 * p
        fire(j0 + 1, 1, sem_b)
        drain(0, sem_a)
        accum(j0, 0)

        @pl.when(j0 + 2 < _CH)
        def _():
          fire(j0 + 2, 0, sem_a)

        drain(1, sem_b)
        accum(j0 + 1, 1)
        return carry2

      lax.fori_loop(0, _CH // 2, pair_body, 0)
      pltpu.sync_copy(pool_v, out_hbm.at[pl.ds(cbase, _CH)])
      return carry

    lax.fori_loop(0, _EPW // _CH, chunk_body, 0)

  return body(xfp, tab_lin)


def _mlp_tc(pooled, W1, b1, W2, b2, W3, b3):
  bb = 2048

  def body(p_ref, w1_ref, b1_ref, w2_ref, b2_ref, w3_ref, b3_ref, o_ref):
    h = jnp.dot(p_ref[...], w1_ref[...], preferred_element_type=jnp.float32)
    h = jnp.maximum(h + b1_ref[...], 0.0)
    h = jnp.dot(h, w2_ref[...], preferred_element_type=jnp.float32)
    h = jnp.maximum(h + b2_ref[...], 0.0)
    o_ref[...] = (
        jnp.dot(h, w3_ref[...], preferred_element_type=jnp.float32)
        + b3_ref[...]
    )

  return pl.pallas_call(
      body,
      grid=(_B // bb,),
      in_specs=[
          pl.BlockSpec((bb, _D), lambda i: (i, 0)),
          pl.BlockSpec((_D, _H), lambda i: (0, 0)),
          pl.BlockSpec((1, _H), lambda i: (0, 0)),
          pl.BlockSpec((_H, _H), lambda i: (0, 0)),
          pl.BlockSpec((1, _H), lambda i: (0, 0)),
          pl.BlockSpec((_H, 1), lambda i: (0, 0)),
          pl.BlockSpec((1, 1), lambda i: (0, 0)),
      ],
      out_specs=pl.BlockSpec((bb, 1), lambda i: (i, 0)),
      out_shape=jax.ShapeDtypeStruct((_B, 1), jnp.float32),
  )(pooled, W1, b1.reshape(1, _H), W2, b2.reshape(1, _H), W3,
    b3.reshape(1, 1))


@jax.jit
def _run(x, table, W1, b1, W2, b2, W3, b3):
  packed = _repack_table_tc(table.T)
  tab_lin = packed.reshape(4 * _NBLK * _BP, _D)
  xfp = _repack_x_sc(x.astype(jnp.int32).T).reshape(2, _B, 128)
  pooled = _pool_sc(xfp, tab_lin)
  return _mlp_tc(pooled, W1, b1, W2, b2, W3, b3)


def kernel(x, table, W1, b1, W2, b2, W3, b3):
  return _run(x, table, W1, b1, W2, b2, W3, b3)


# speedup vs baseline: 2.2222x; 1.1768x over previous
import functools

import jax
import jax.numpy as jnp
from jax import lax
from jax.experimental import pallas as pl
from jax.experimental.pallas import tpu as pltpu
from jax.experimental.pallas import tpu_sc as plsc

_B, _L, _V, _D = 16384, 200, 1000000, 32
_H = 150
_NC, _NS = 2, 16
_NW = _NC * _NS
_EPW = _B // _NW
_CH = 64
_BV = 8192
_BP = _BV // 4
_NBLK = (_V + _BV - 1) // _BV
_L0, _L1 = 128, 72

_MESH = dict(core_axis_name="c", subcore_axis_name="s")


def _phi(v):
  return (v & -_BV) | ((v & (_BP - 1)) << 2) | ((v >> 11) & 3)


def _repack_table_tc(tabT):
  def body(i_ref, o_ref):
    for c in range(4):
      o_ref[:, 32 * c:32 * (c + 1)] = jnp.transpose(
          i_ref[:, _BP * c:_BP * (c + 1)])

  return pl.pallas_call(
      body,
      grid=(_NBLK,),
      in_specs=[pl.BlockSpec((_D, _BV), lambda i: (0, i))],
      out_specs=pl.BlockSpec((_BP, 128), lambda i: (i, 0)),
      out_shape=jax.ShapeDtypeStruct((_NBLK * _BP, 128), jnp.float32),
  )(tabT)


def _repack_x_sc(xT):
  @functools.partial(
      pl.kernel,
      mesh=plsc.VectorSubcoreMesh(**_MESH),
      compiler_params=pltpu.CompilerParams(
          use_tc_tiling_on_sc=False, needs_layout_passes=False),
      out_type=jax.ShapeDtypeStruct((2 * _B, 128), jnp.int32),
      scratch_types=[
          pltpu.VMEM((2, _L, 128), jnp.int32),
          pltpu.VMEM((128, 128), jnp.int32),
          pltpu.SemaphoreType.DMA,
      ],
  )
  def body(x_hbm, xfp_hbm, xin, xout, sem_x):
    wid = lax.axis_index("s") * _NC + lax.axis_index("c")
    iota = lax.iota(jnp.int32, 16)

    def x_in(m, buf):
      pltpu.async_copy(
          x_hbm.at[:, pl.ds((wid + _NW * m) * 128, 128)], xin.at[buf],
          sem_x)

    def x_in_wait(buf):
      pltpu.make_async_copy(
          x_hbm.at[:, pl.ds(0, 128)], xin.at[buf], sem_x).wait()

    def x_block(m, buf):
      bb = wid + _NW * m
      for h in range(2):
        npiece = 8 if h == 0 else 5

        def xrow(b, carry):
          col = jnp.full((16,), b, jnp.int32)
          for k in range(npiece):
            ls = jnp.minimum(128 * h + 16 * k + iota, _L - 1)
            vals = plsc.load_gather(xin.at[buf], [ls, col])
            xout[b, pl.ds(16 * k, 16)] = _phi(vals)
          return carry

        lax.fori_loop(0, 128, xrow, 0, unroll=4)
        pltpu.sync_copy(xout, xfp_hbm.at[pl.ds(h * _B + bb * 128, 128)])

    x_in(0, 0)
    for m in range(4):
      if m + 1 < 4:
        x_in(m + 1, (m + 1) % 2)
      x_in_wait(m % 2)
      x_block(m, m % 2)

  return body(xT)


def _pool_sc(xfp, tab_lin):
  @functools.partial(
      pl.kernel,
      mesh=plsc.VectorSubcoreMesh(**_MESH),
      compiler_params=pltpu.CompilerParams(use_tc_tiling_on_sc=False),
      out_type=jax.ShapeDtypeStruct((_B, _D), jnp.float32),
      scratch_types=[
          pltpu.VMEM((2, _CH, 128), jnp.int32),
          pltpu.VMEM((3, _L0, _D), jnp.float32),
          pltpu.VMEM((3, _L1, _D), jnp.float32),
          pltpu.VMEM((_CH, _D), jnp.float32),
          pltpu.SemaphoreType.DMA,
          pltpu.SemaphoreType.DMA,
          pltpu.SemaphoreType.DMA,
      ],
  )
  def body(x_hbm, tab_hbm, out_hbm, idx_v, rows_a, rows_b, pool_v,
           sem_a, sem_b, sem_c):
    wid = lax.axis_index("s") * _NC + lax.axis_index("c")
    base = wid * _EPW

    def fire(j, buf, sem):
      pltpu.async_copy(tab_hbm.at[idx_v.at[0, j]], rows_a.at[buf], sem)
      pltpu.async_copy(tab_hbm.at[idx_v.at[1, j, pl.ds(0, _L1)]],
                       rows_b.at[buf], sem)

    def drain(buf, sem):
      pltpu.make_async_copy(
          tab_hbm.at[idx_v.at[0, 0]], rows_a.at[buf], sem).wait()
      pltpu.make_async_copy(
          tab_hbm.at[idx_v.at[1, 0, pl.ds(0, _L1)]], rows_b.at[buf],
          sem).wait()

    def accum(j, buf):
      def row2_body(r, accs):
        a0, a1 = accs
        a0 = a0 + rows_a[buf, r, pl.ds(0, 16)]
        a0 = a0 + rows_b[buf, r, pl.ds(0, 16)]
        a1 = a1 + rows_a[buf, r, pl.ds(16, 16)]
        a1 = a1 + rows_b[buf, r, pl.ds(16, 16)]
        return (a0, a1)

      def row1_body(r, accs):
        a0, a1 = accs
        a0 = a0 + rows_a[buf, r, pl.ds(0, 16)]
        a1 = a1 + rows_a[buf, r, pl.ds(16, 16)]
        return (a0, a1)

      z = jnp.zeros((16,), jnp.float32)
      accs = lax.fori_loop(0, _L1, row2_body, (z, z))
      a0, a1 = lax.fori_loop(_L1, _L0, row1_body, accs)
      pool_v[j, pl.ds(0, 16)] = a0 * (1.0 / _L)
      pool_v[j, pl.ds(16, 16)] = a1 * (1.0 / _L)

    def chunk_body(ci, carry):
      cbase = base + ci * _CH
      pltpu.sync_copy(x_hbm.at[0, pl.ds(cbase, _CH)], idx_v.at[0])
      pltpu.sync_copy(x_hbm.at[1, pl.ds(cbase, _CH)], idx_v.at[1])
      fire(0, 0, sem_a)
      fire(1, 1, sem_b)

      def tri_body(p, carry2):
        j0 = 3 * p
        fire(j0 + 2, 2, sem_c)
        drain(0, sem_a)
        accum(j0, 0)

        @pl.when(j0 + 3 < _CH)
        def _():
          fire(j0 + 3, 0, sem_a)

        drain(1, sem_b)
        accum(j0 + 1, 1)

        @pl.when(j0 + 4 < _CH)
        def _():
          fire(j0 + 4, 1, sem_b)

        drain(2, sem_c)
        accum(j0 + 2, 2)
        return carry2

      lax.fori_loop(0, _CH // 3, tri_body, 0)
      drain(0, sem_a)
      accum(_CH - 1, 0)
      pltpu.sync_copy(pool_v, out_hbm.at[pl.ds(cbase, _CH)])
      return carry

    lax.fori_loop(0, _EPW // _CH, chunk_body, 0)

  return body(xfp, tab_lin)


def _mlp_tc(pooled, W1, b1, W2, b2, W3, b3):
  bb = 2048

  def body(p_ref, w1_ref, b1_ref, w2_ref, b2_ref, w3_ref, b3_ref, o_ref):
    h = jnp.dot(p_ref[...], w1_ref[...], preferred_element_type=jnp.float32)
    h = jnp.maximum(h + b1_ref[...], 0.0)
    h = jnp.dot(h, w2_ref[...], preferred_element_type=jnp.float32)
    h = jnp.maximum(h + b2_ref[...], 0.0)
    o_ref[...] = (
        jnp.dot(h, w3_ref[...], preferred_element_type=jnp.float32)
        + b3_ref[...]
    )

  return pl.pallas_call(
      body,
      grid=(_B // bb,),
      in_specs=[
          pl.BlockSpec((bb, _D), lambda i: (i, 0)),
          pl.BlockSpec((_D, _H), lambda i: (0, 0)),
          pl.BlockSpec((1, _H), lambda i: (0, 0)),
          pl.BlockSpec((_H, _H), lambda i: (0, 0)),
          pl.BlockSpec((1, _H), lambda i: (0, 0)),
          pl.BlockSpec((_H, 1), lambda i: (0, 0)),
          pl.BlockSpec((1, 1), lambda i: (0, 0)),
      ],
      out_specs=pl.BlockSpec((bb, 1), lambda i: (i, 0)),
      out_shape=jax.ShapeDtypeStruct((_B, 1), jnp.float32),
  )(pooled, W1, b1.reshape(1, _H), W2, b2.reshape(1, _H), W3,
    b3.reshape(1, 1))


@jax.jit
def _run(x, table, W1, b1, W2, b2, W3, b3):
  packed = _repack_table_tc(table.T)
  tab_lin = packed.reshape(4 * _NBLK * _BP, _D)
  xfp = _repack_x_sc(x.astype(jnp.int32).T).reshape(2, _B, 128)
  pooled = _pool_sc(xfp, tab_lin)
  return _mlp_tc(pooled, W1, b1, W2, b2, W3, b3)


def kernel(x, table, W1, b1, W2, b2, W3, b3):
  return _run(x, table, W1, b1, W2, b2, W3, b3)


# speedup vs baseline: 2.2420x; 1.0089x over previous
import functools

import jax
import jax.numpy as jnp
from jax import lax
from jax.experimental import pallas as pl
from jax.experimental.pallas import tpu as pltpu
from jax.experimental.pallas import tpu_sc as plsc

_B, _L, _V, _D = 16384, 200, 1000000, 32
_H = 150
_NC, _NS = 2, 16
_NW = _NC * _NS
_EPW = _B // _NW
_CH = 64
_BV = 8192
_BP = _BV // 4
_NBLK = (_V + _BV - 1) // _BV
_L0, _L1 = 128, 72

_MESH = dict(core_axis_name="c", subcore_axis_name="s")


def _phi(v):
  return (v & -_BV) | ((v & (_BP - 1)) << 2) | ((v >> 11) & 3)


def _repack_table_tc(tabT):
  def body(i_ref, o_ref):
    for c in range(4):
      o_ref[:, 32 * c:32 * (c + 1)] = jnp.transpose(
          i_ref[:, _BP * c:_BP * (c + 1)])

  return pl.pallas_call(
      body,
      grid=(_NBLK,),
      in_specs=[pl.BlockSpec((_D, _BV), lambda i: (0, i))],
      out_specs=pl.BlockSpec((_BP, 128), lambda i: (i, 0)),
      out_shape=jax.ShapeDtypeStruct((_NBLK * _BP, 128), jnp.float32),
  )(tabT)


def _repack_x_sc(xT):
  @functools.partial(
      pl.kernel,
      mesh=plsc.VectorSubcoreMesh(**_MESH),
      compiler_params=pltpu.CompilerParams(
          use_tc_tiling_on_sc=False, needs_layout_passes=False),
      out_type=jax.ShapeDtypeStruct((2 * _B, 128), jnp.int32),
      scratch_types=[
          pltpu.VMEM((2, _L, 128), jnp.int32),
          pltpu.VMEM((128, 128), jnp.int32),
          pltpu.SemaphoreType.DMA,
      ],
  )
  def body(x_hbm, xfp_hbm, xin, xout, sem_x):
    wid = lax.axis_index("s") * _NC + lax.axis_index("c")
    iota = lax.iota(jnp.int32, 16)

    def x_in(m, buf):
      pltpu.async_copy(
          x_hbm.at[:, pl.ds((wid + _NW * m) * 128, 128)], xin.at[buf],
          sem_x)

    def x_in_wait(buf):
      pltpu.make_async_copy(
          x_hbm.at[:, pl.ds(0, 128)], xin.at[buf], sem_x).wait()

    def x_block(m, buf):
      bb = wid + _NW * m
      for h in range(2):
        npiece = 8 if h == 0 else 5

        def xrow(b, carry):
          col = jnp.full((16,), b, jnp.int32)
          for k in range(npiece):
            ls = jnp.minimum(128 * h + 16 * k + iota, _L - 1)
            vals = plsc.load_gather(xin.at[buf], [ls, col])
            xout[b, pl.ds(16 * k, 16)] = _phi(vals)
          return carry

        lax.fori_loop(0, 128, xrow, 0, unroll=4)
        pltpu.sync_copy(xout, xfp_hbm.at[pl.ds(h * _B + bb * 128, 128)])

    x_in(0, 0)
    for m in range(4):
      if m + 1 < 4:
        x_in(m + 1, (m + 1) % 2)
      x_in_wait(m % 2)
      x_block(m, m % 2)

  return body(xT)


def _pool_sc(xfp, tab_lin):
  @functools.partial(
      pl.kernel,
      mesh=plsc.VectorSubcoreMesh(**_MESH),
      compiler_params=pltpu.CompilerParams(use_tc_tiling_on_sc=False),
      out_type=jax.ShapeDtypeStruct((_B, _D), jnp.float32),
      scratch_types=[
          pltpu.VMEM((2, _CH, 128), jnp.int32),
          pltpu.VMEM((4, _L0, _D), jnp.float32),
          pltpu.VMEM((4, _L1, _D), jnp.float32),
          pltpu.VMEM((_CH, _D), jnp.float32),
          pltpu.SemaphoreType.DMA,
          pltpu.SemaphoreType.DMA,
          pltpu.SemaphoreType.DMA,
          pltpu.SemaphoreType.DMA,
      ],
  )
  def body(x_hbm, tab_hbm, out_hbm, idx_v, rows_a, rows_b, pool_v,
           sem_a, sem_b, sem_c, sem_d):
    wid = lax.axis_index("s") * _NC + lax.axis_index("c")
    base = wid * _EPW

    def fire(j, buf, sem):
      pltpu.async_copy(tab_hbm.at[idx_v.at[0, j]], rows_a.at[buf], sem)
      pltpu.async_copy(tab_hbm.at[idx_v.at[1, j, pl.ds(0, _L1)]],
                       rows_b.at[buf], sem)

    def drain(buf, sem):
      pltpu.make_async_copy(
          tab_hbm.at[idx_v.at[0, 0]], rows_a.at[buf], sem).wait()
      pltpu.make_async_copy(
          tab_hbm.at[idx_v.at[1, 0, pl.ds(0, _L1)]], rows_b.at[buf],
          sem).wait()

    def accum(j, buf):
      def row2_body(r, accs):
        a0, a1 = accs
        a0 = a0 + rows_a[buf, r, pl.ds(0, 16)]
        a0 = a0 + rows_b[buf, r, pl.ds(0, 16)]
        a1 = a1 + rows_a[buf, r, pl.ds(16, 16)]
        a1 = a1 + rows_b[buf, r, pl.ds(16, 16)]
        return (a0, a1)

      def row1_body(r, accs):
        a0, a1 = accs
        a0 = a0 + rows_a[buf, r, pl.ds(0, 16)]
        a1 = a1 + rows_a[buf, r, pl.ds(16, 16)]
        return (a0, a1)

      z = jnp.zeros((16,), jnp.float32)
      accs = lax.fori_loop(0, _L1, row2_body, (z, z))
      a0, a1 = lax.fori_loop(_L1, _L0, row1_body, accs)
      pool_v[j, pl.ds(0, 16)] = a0 * (1.0 / _L)
      pool_v[j, pl.ds(16, 16)] = a1 * (1.0 / _L)

    def chunk_body(ci, carry):
      cbase = base + ci * _CH
      pltpu.sync_copy(x_hbm.at[0, pl.ds(cbase, _CH)], idx_v.at[0])
      pltpu.sync_copy(x_hbm.at[1, pl.ds(cbase, _CH)], idx_v.at[1])
      fire(0, 0, sem_a)
      fire(1, 1, sem_b)
      fire(2, 2, sem_c)

      def quad_body(p, carry2):
        j0 = 4 * p
        fire(j0 + 3, 3, sem_d)
        drain(0, sem_a)
        accum(j0, 0)

        @pl.when(j0 + 4 < _CH)
        def _():
          fire(j0 + 4, 0, sem_a)

        drain(1, sem_b)
        accum(j0 + 1, 1)

        @pl.when(j0 + 5 < _CH)
        def _():
          fire(j0 + 5, 1, sem_b)

        drain(2, sem_c)
        accum(j0 + 2, 2)

        @pl.when(j0 + 6 < _CH)
        def _():
          fire(j0 + 6, 2, sem_c)

        drain(3, sem_d)
        accum(j0 + 3, 3)
        return carry2

      lax.fori_loop(0, _CH // 4, quad_body, 0)
      pltpu.sync_copy(pool_v, out_hbm.at[pl.ds(cbase, _CH)])
      return carry

    lax.fori_loop(0, _EPW // _CH, chunk_body, 0)

  return body(xfp, tab_lin)


def _mlp_tc(pooled, W1, b1, W2, b2, W3, b3):
  bb = 2048

  def body(p_ref, w1_ref, b1_ref, w2_ref, b2_ref, w3_ref, b3_ref, o_ref):
    h = jnp.dot(p_ref[...], w1_ref[...], preferred_element_type=jnp.float32)
    h = jnp.maximum(h + b1_ref[...], 0.0)
    h = jnp.dot(h, w2_ref[...], preferred_element_type=jnp.float32)
    h = jnp.maximum(h + b2_ref[...], 0.0)
    o_ref[...] = (
        jnp.dot(h, w3_ref[...], preferred_element_type=jnp.float32)
        + b3_ref[...]
    )

  return pl.pallas_call(
      body,
      grid=(_B // bb,),
      in_specs=[
          pl.BlockSpec((bb, _D), lambda i: (i, 0)),
          pl.BlockSpec((_D, _H), lambda i: (0, 0)),
          pl.BlockSpec((1, _H), lambda i: (0, 0)),
          pl.BlockSpec((_H, _H), lambda i: (0, 0)),
          pl.BlockSpec((1, _H), lambda i: (0, 0)),
          pl.BlockSpec((_H, 1), lambda i: (0, 0)),
          pl.BlockSpec((1, 1), lambda i: (0, 0)),
      ],
      out_specs=pl.BlockSpec((bb, 1), lambda i: (i, 0)),
      out_shape=jax.ShapeDtypeStruct((_B, 1), jnp.float32),
  )(pooled, W1, b1.reshape(1, _H), W2, b2.reshape(1, _H), W3,
    b3.reshape(1, 1))


@jax.jit
def _run(x, table, W1, b1, W2, b2, W3, b3):
  packed = _repack_table_tc(table.T)
  tab_lin = packed.reshape(4 * _NBLK * _BP, _D)
  xfp = _repack_x_sc(x.astype(jnp.int32).T).reshape(2, _B, 128)
  pooled = _pool_sc(xfp, tab_lin)
  return _mlp_tc(pooled, W1, b1, W2, b2, W3, b3)


def kernel(x, table, W1, b1, W2, b2, W3, b3):
  return _run(x, table, W1, b1, W2, b2, W3, b3)
